# baseline (speedup 1.0000x reference)
import jax
import jax.numpy as jnp
from jax import lax
from jax.experimental import pallas as pl
from jax.experimental.pallas import tpu as pltpu
from jax.experimental.pallas import tpu_sc as plsc

NUM_FIELDS = 26
VOCAB = 100000
DIM = 64
BATCH = 16384
N = BATCH * NUM_FIELDS

_NC, _NS = 2, 16
NW = _NC * _NS
PER_W = N // NW
SUB = 128
N_SUB = PER_W // SUB
GROUP = 8
N_GROUP = N_SUB // GROUP
GROUP_ROWS = GROUP * SUB


def _body(ftab_hbm, idx_hbm, out_hbm, idx_v, rows_v, sem):
    wid = lax.axis_index("s") * _NC + lax.axis_index("c")
    base = wid * PER_W

    pltpu.sync_copy(idx_hbm.at[pl.ds(base, PER_W)], idx_v)

    lanes = lax.iota(jnp.int32, 16)

    def fix(j, _):
        pos = j * 16 + lanes
        vec = idx_v[pl.ds(j * 16, 16)]
        idx_v[pl.ds(j * 16, 16)] = vec + (pos % NUM_FIELDS) * VOCAB
        return 0

    lax.fori_loop(0, PER_W // 16, fix, 0)

    def group(g, _):
        handles = []
        for k in range(GROUP):
            src = ftab_hbm.at[idx_v.at[pl.ds((g * GROUP + k) * SUB, SUB)]]
            dst = rows_v.at[pl.ds(k * SUB, SUB)]
            handles.append(pltpu.async_copy(src, dst, sem))
        for h in handles:
            h.wait()
        pltpu.sync_copy(rows_v, out_hbm.at[pl.ds(base + g * GROUP_ROWS, GROUP_ROWS)])
        return 0

    lax.fori_loop(0, N_GROUP, group, 0)


def kernel(indices, tables):
    ftab = tables.reshape(NUM_FIELDS * VOCAB, DIM)
    idx_flat = indices.reshape(N).astype(jnp.int32)

    mesh = plsc.VectorSubcoreMesh(core_axis_name="c", subcore_axis_name="s")
    out = pl.kernel(
        _body,
        out_type=jax.ShapeDtypeStruct((N, DIM), jnp.float32),
        mesh=mesh,
        scratch_types=[
            pltpu.VMEM((PER_W,), jnp.int32),
            pltpu.VMEM((GROUP_ROWS, DIM), jnp.float32),
            pltpu.SemaphoreType.DMA,
        ],
        compiler_params=pltpu.CompilerParams(use_tc_tiling_on_sc=False),
    )(ftab, idx_flat)
    return out.reshape(BATCH, NUM_FIELDS * DIM)


# speedup vs baseline: 2.7840x; 2.7840x over previous
import jax
import jax.numpy as jnp
from jax import lax
from jax.experimental import pallas as pl
from jax.experimental.pallas import tpu as pltpu
from jax.experimental.pallas import tpu_sc as plsc

NUM_FIELDS = 26
VOCAB = 100000
DIM = 64
BATCH = 16384
R = NUM_FIELDS * DIM

_NC, _NS = 2, 16
NW = _NC * _NS
ROWS_PER_W = R // NW
OUT_CHUNK = BATCH // 2
VEC = 16
UNROLL = 8


def _body(p_hbm, idx_hbm, out_hbm, row_v, idx_v, out_v, sem):
    wid = lax.axis_index("s") * _NC + lax.axis_index("c")
    r0 = wid * ROWS_PER_W

    def do_row(i, prev_f):
        r = r0 + i
        f = r // DIM

        @pl.when(jnp.logical_or(i == 0, f != prev_f))
        def _():
            pltpu.sync_copy(idx_hbm.at[f], idx_v)

        pltpu.sync_copy(p_hbm.at[r], row_v)

        for h in range(BATCH // OUT_CHUNK):
            def gather_step(j, _):
                base = h * OUT_CHUNK + j * (VEC * UNROLL)
                for u in range(UNROLL):
                    iv = idx_v[pl.ds(base + u * VEC, VEC)]
                    g = plsc.load_gather(row_v, [iv])
                    out_v[pl.ds(j * (VEC * UNROLL) + u * VEC, VEC)] = g
                return 0

            lax.fori_loop(0, OUT_CHUNK // (VEC * UNROLL), gather_step, 0)
            pltpu.sync_copy(out_v, out_hbm.at[r, pl.ds(h * OUT_CHUNK, OUT_CHUNK)])
        return f

    lax.fori_loop(0, ROWS_PER_W, do_row, jnp.int32(-1))


def kernel(indices, tables):
    p = jnp.transpose(tables, (0, 2, 1)).reshape(R, VOCAB)
    idx_t = jnp.transpose(indices.astype(jnp.int32), (1, 0))

    mesh = plsc.VectorSubcoreMesh(core_axis_name="c", subcore_axis_name="s")
    out_t = pl.kernel(
        _body,
        out_type=jax.ShapeDtypeStruct((R, BATCH), jnp.float32),
        mesh=mesh,
        scratch_types=[
            pltpu.VMEM((VOCAB,), jnp.float32),
            pltpu.VMEM((BATCH,), jnp.int32),
            pltpu.VMEM((OUT_CHUNK,), jnp.float32),
            pltpu.SemaphoreType.DMA,
        ],
        compiler_params=pltpu.CompilerParams(needs_layout_passes=False),
    )(p, idx_t)
    return out_t.T.reshape(BATCH, NUM_FIELDS * DIM)
